# baseline (speedup 1.0000x reference)
import functools

import jax
import jax.numpy as jnp
from jax import lax
from jax.experimental import pallas as pl
from jax.experimental.pallas import tpu as pltpu
from jax.experimental.pallas import tpu_sc as plsc

HIDDEN = 1024
NC = 2
NS = 16
NW = NC * NS
CHUNK = 32


def _make_gather(ntok: int):
    assert ntok % NW == 0
    bpw = ntok // NW
    assert bpw % CHUNK == 0
    nch = bpw // CHUNK

    mesh = plsc.VectorSubcoreMesh(core_axis_name="c", subcore_axis_name="s")

    nbuf = 3
    assert nch >= 4

    @functools.partial(
        pl.kernel,
        mesh=mesh,
        out_type=jax.ShapeDtypeStruct((ntok, HIDDEN), jnp.float32),
        scratch_types=[
            pltpu.VMEM((bpw,), jnp.int32),
            [pltpu.VMEM((CHUNK, HIDDEN), jnp.float32) for _ in range(nbuf)],
            [pltpu.SemaphoreType.DMA for _ in range(nbuf)],
            [pltpu.SemaphoreType.DMA for _ in range(nbuf)],
        ],
    )
    def gather_kernel(ids_hbm, table_hbm, out_hbm, idx_v, bufs, gsems, ssems):
        wid = lax.axis_index("s") * NC + lax.axis_index("c")
        base = wid * bpw
        pltpu.sync_copy(ids_hbm.at[pl.ds(base, bpw)], idx_v)

        def start_gather(ch, b):
            pltpu.async_copy(
                table_hbm.at[idx_v.at[pl.ds(ch * CHUNK, CHUNK)]], bufs[b], gsems[b]
            )

        def wait_gather(b):
            pltpu.make_async_copy(
                table_hbm.at[idx_v.at[pl.ds(0, CHUNK)]], bufs[b], gsems[b]
            ).wait()

        def start_store(ch, b):
            pltpu.async_copy(
                bufs[b], out_hbm.at[pl.ds(base + ch * CHUNK, CHUNK)], ssems[b]
            )

        def wait_store(b):
            pltpu.make_async_copy(
                bufs[b], out_hbm.at[pl.ds(base, CHUNK)], ssems[b]
            ).wait()

        start_gather(0, 0)
        start_gather(1, 1)
        wait_gather(0)
        start_store(0, 0)
        start_gather(2, 2)

        body = nch - nch % 3

        def step(ch, b):
            wait_gather(b)
            start_store(ch, b)
            wait_store((b + 2) % 3)
            pltpu.async_copy(
                table_hbm.at[idx_v.at[pl.ds((ch + 2) * CHUNK, CHUNK)]],
                bufs[(b + 2) % 3],
                gsems[(b + 2) % 3],
            )

        @pl.loop(1, body + 1, step=3)
        def _(i):
            for b_off in range(3):
                ch = i + b_off
                b = (1 + b_off) % 3

                @pl.when(ch + 2 < nch)
                def _():
                    step(ch, b)

        for ch in range(nch - 2, nch):
            b = ch % 3
            wait_gather(b)
            start_store(ch, b)
            wait_store((b + 2) % 3)
        wait_store((nch - 1) % 3)

    return gather_kernel


def kernel(input_ids, position_ids, embed_tokens):
    batch, seq = input_ids.shape
    ids_flat = input_ids.reshape(-1)
    rows = _make_gather(batch * seq)(ids_flat, embed_tokens)
    hidden_states = rows.reshape(batch, seq, HIDDEN)
    return hidden_states, position_ids
